# baseline (speedup 1.0000x reference)
import jax
import jax.numpy as jnp
import numpy as np
from jax import lax
from jax.experimental import pallas as pl

N = 4096
H = 256
W = 256
C = 3
CHUNK = 128
ROWS = 8
NCHUNKS = N // CHUNK
PX = ROWS * W


def _raster_kernel(cx_ref, cy_ref, a_ref, b_ref, c_ref, vt_ref, out_ref):
    i = pl.program_id(0)
    pix = lax.broadcasted_iota(jnp.int32, (CHUNK, PX), 1)
    gx = (pix & (W - 1)).astype(jnp.float32)
    gy = (pix >> 8).astype(jnp.float32) + (i * ROWS).astype(jnp.float32)

    def body(j, acc):
        cxc = cx_ref[j].reshape(CHUNK, 1)
        cyc = cy_ref[j].reshape(CHUNK, 1)
        ac = a_ref[j].reshape(CHUNK, 1)
        bc = b_ref[j].reshape(CHUNK, 1)
        cc = c_ref[j].reshape(CHUNK, 1)
        dx = gx - cxc
        dy = gy - cyc
        power = (ac * dx) * dx + ((bc * dx) + (cc * dy)) * dy
        w = jnp.exp(power)
        v = vt_ref[:, pl.ds(j * CHUNK, CHUNK)]
        return acc + jnp.dot(v, w, preferred_element_type=jnp.float32)

    acc = lax.fori_loop(0, NCHUNKS, body, jnp.zeros((C, PX), jnp.float32))
    out_ref[...] = acc.reshape(C, ROWS, W)


def kernel(xy, scaling, rotation, values, opacity):
    xy_t = jnp.tanh(xy)
    s = jnp.abs(scaling) + 0.3
    theta = jax.nn.sigmoid(rotation[:, 0]) * 2.0 * np.pi
    cos_t = jnp.cos(theta)
    sin_t = jnp.sin(theta)
    s0 = s[:, 0]
    s1 = s[:, 1]
    a = cos_t * cos_t * s0 * s0 + sin_t * sin_t * s1 * s1
    b = cos_t * sin_t * (s0 * s0 - s1 * s1)
    c = sin_t * sin_t * s0 * s0 + cos_t * cos_t * s1 * s1
    det = a * c - b * b
    conic_a = c / det
    conic_b = -b / det
    conic_c = a / det
    cx = 0.5 * W * (xy_t[:, 0] + 1.0) - 0.5
    cy = 0.5 * H * (xy_t[:, 1] + 1.0) - 0.5

    q = lambda x: x.reshape(NCHUNKS, CHUNK)
    vt = (values * opacity).T

    grid = H // ROWS
    full = lambda shp: pl.BlockSpec(shp, lambda i: tuple(0 for _ in shp))
    out = pl.pallas_call(
        _raster_kernel,
        grid=(grid,),
        in_specs=[full((NCHUNKS, CHUNK))] * 5 + [full((C, N))],
        out_specs=pl.BlockSpec((C, ROWS, W), lambda i: (0, i, 0)),
        out_shape=jax.ShapeDtypeStruct((C, H, W), jnp.float32),
    )(q(cx), q(cy), q(-0.5 * conic_a), q(-conic_b), q(-0.5 * conic_c), vt)

    return out.reshape(1, C, H, W)


# speedup vs baseline: 5.2521x; 5.2521x over previous
import functools
import jax
import jax.numpy as jnp
import numpy as np
from jax import lax
from jax.experimental import pallas as pl
from jax.experimental.pallas import tpu as pltpu

N = 4096
H = 256
W = 256
C = 3
G = 64
ROWS = 8
NCH = N // G
PX = ROWS * W
RCUT = 8.5


def _raster_kernel(bounds_ref, cx_ref, cy_ref, a_ref, b_ref, c_ref, vt_ref,
                   out_ref):
    i = pl.program_id(0)
    lo = bounds_ref[0, i]
    hi = bounds_ref[1, i]
    pix = lax.broadcasted_iota(jnp.int32, (G, PX), 1)
    gx = (pix & (W - 1)).astype(jnp.float32)
    gy = (pix >> 8).astype(jnp.float32) + (i * ROWS).astype(jnp.float32)

    def body(j, acc):
        cxc = cx_ref[j].reshape(G, 1)
        cyc = cy_ref[j].reshape(G, 1)
        ac = a_ref[j].reshape(G, 1)
        bc = b_ref[j].reshape(G, 1)
        cc = c_ref[j].reshape(G, 1)
        dx = gx - cxc
        dy = gy - cyc
        power = (ac * dx) * dx + ((bc * dx) + (cc * dy)) * dy
        w = jnp.exp(power)
        return acc + jnp.dot(vt_ref[:, j], w, preferred_element_type=jnp.float32)

    acc = lax.fori_loop(lo, hi, body, jnp.zeros((C, PX), jnp.float32))
    out_ref[...] = acc.reshape(C, ROWS, W)


def kernel(xy, scaling, rotation, values, opacity):
    xy_t = jnp.tanh(xy)
    s = jnp.abs(scaling) + 0.3
    theta = jax.nn.sigmoid(rotation[:, 0]) * 2.0 * np.pi
    cos_t = jnp.cos(theta)
    sin_t = jnp.sin(theta)
    s0 = s[:, 0]
    s1 = s[:, 1]
    a = cos_t * cos_t * s0 * s0 + sin_t * sin_t * s1 * s1
    b = cos_t * sin_t * (s0 * s0 - s1 * s1)
    c = sin_t * sin_t * s0 * s0 + cos_t * cos_t * s1 * s1
    det = a * c - b * b
    cx = 0.5 * W * (xy_t[:, 0] + 1.0) - 0.5
    cy = 0.5 * H * (xy_t[:, 1] + 1.0) - 0.5
    vop = values * opacity

    cy_s, cx_s, a_s, b_s, c_s, v0, v1, v2 = lax.sort(
        (cy, cx, -0.5 * c / det, b / det, -0.5 * a / det,
         vop[:, 0], vop[:, 1], vop[:, 2]), num_keys=1)

    grid = H // ROWS
    starts = jnp.searchsorted(cy_s, jnp.arange(grid) * ROWS - RCUT)
    ends = jnp.searchsorted(cy_s, jnp.arange(grid) * ROWS + (ROWS - 1) + RCUT,
                            side='right')
    bounds = jnp.stack([starts // G, (ends + G - 1) // G]).astype(jnp.int32)

    q = lambda x: x.reshape(NCH, G)
    vt = jnp.stack([v0, v1, v2]).reshape(C, NCH, G)

    full = lambda shp: pl.BlockSpec(shp, lambda *_: tuple(0 for _ in shp))
    out = pl.pallas_call(
        _raster_kernel,
        grid_spec=pltpu.PrefetchScalarGridSpec(
            num_scalar_prefetch=1,
            grid=(grid,),
            in_specs=[full((NCH, G))] * 5 + [full((C, NCH, G))],
            out_specs=pl.BlockSpec((C, ROWS, W), lambda i, b: (0, i, 0)),
        ),
        out_shape=jax.ShapeDtypeStruct((C, H, W), jnp.float32),
    )(bounds, q(cx_s), q(cy_s), q(a_s), q(b_s), q(c_s), vt)

    return out.reshape(1, C, H, W)
